# Initial kernel scaffold; baseline (speedup 1.0000x reference)
import jax
import jax.numpy as jnp
from jax.experimental import pallas as pl


def kernel(pos, h, u, edge_index, W1, b1, W2, b2, U1, ub1, U2, ub2):
    raise NotImplementedError("write your pallas kernel here")


# speedup vs baseline: 5.0547x; 5.0547x over previous
import functools

import jax
import jax.numpy as jnp
from jax import lax
from jax.experimental import pallas as pl
from jax.experimental.pallas import tpu as pltpu
from jax.experimental.pallas import tpu_sc as plsc

NC = 2
NS = 16
NW = NC * NS
C = 128


def _stage1(x1, pos, w1s, w1d, b1):
    n, k1 = x1.shape
    blk = 4000 if n % 4000 == 0 else n
    grid = n // blk

    def body(x1_ref, pos_ref, w1s_ref, w1d_ref, b1_ref, a_ref, bd_ref):
        a_ref[...] = (
            jnp.dot(x1_ref[...], w1s_ref[...], preferred_element_type=jnp.float32)
            + b1_ref[...]
        )
        bd_ref[...] = jnp.dot(
            pos_ref[...], w1d_ref[...], preferred_element_type=jnp.float32
        )

    return pl.pallas_call(
        body,
        grid=(grid,),
        in_specs=[
            pl.BlockSpec((blk, k1), lambda i: (i, 0)),
            pl.BlockSpec((blk, 2), lambda i: (i, 0)),
            pl.BlockSpec(w1s.shape, lambda i: (0, 0)),
            pl.BlockSpec(w1d.shape, lambda i: (0, 0)),
            pl.BlockSpec((1, 32), lambda i: (0, 0)),
        ],
        out_specs=[
            pl.BlockSpec((blk, 32), lambda i: (i, 0)),
            pl.BlockSpec((blk, 32), lambda i: (i, 0)),
        ],
        out_shape=[
            jax.ShapeDtypeStruct((n, 32), jnp.float32),
            jax.ShapeDtypeStruct((n, 32), jnp.float32),
        ],
    )(x1, pos, w1s, w1d, b1)


def _edge_body(nchunk, n, acc_rows, a_hbm, b_hbm, sdx_hbm, w2_hbm,
               b2_hbm, out_hbm, idx, buf_a, buf_b, msg, w2_v,
               b2_v, acc, sem_a0, sem_a1, sem_b0, sem_b1, sem_s0,
               sem_s1, sem_i0, sem_i1):
    c = lax.axis_index("c")
    s = lax.axis_index("s")
    wid = c * NS + s
    sem_a = (sem_a0, sem_a1)
    sem_b = (sem_b0, sem_b1)
    sem_s = (sem_s0, sem_s1)
    sem_i = (sem_i0, sem_i1)

    pltpu.sync_copy(w2_hbm, w2_v)
    pltpu.sync_copy(b2_hbm, b2_v)

    def zrow(i, _):
        msg[0, i, :] = jnp.zeros((16,), jnp.float32)
        return 0

    lax.fori_loop(0, C, zrow, 0)
    zchunks = acc_rows // NS // C

    def zblk(k, _):
        pltpu.sync_copy(msg.at[0], acc.at[pl.ds(s * (acc_rows // NS) + k * C, C)])
        return 0

    lax.fori_loop(0, zchunks, zblk, 0)
    plsc.subcore_barrier()

    w2_rows = [w2_v[k, :] for k in range(32)]
    b2_reg = b2_v[...]

    for b in range(2):
        pltpu.sync_copy(sdx_hbm.at[wid, b], idx.at[b])
        pltpu.async_copy(a_hbm.at[idx.at[b, 0]], buf_a.at[b], sem_a[b])
        pltpu.async_copy(b_hbm.at[idx.at[b, 1]], buf_b.at[b], sem_b[b])

    def chunk_quad(i, _):
        j0 = i * 4
        for q in range(4):
            j = j0 + q
            p = q % 2
            q2 = (q + 2) % 4
            pltpu.make_async_copy(
                a_hbm.at[idx.at[q, 0]], buf_a.at[p], sem_a[p]).wait()
            pltpu.make_async_copy(
                b_hbm.at[idx.at[q, 1]], buf_b.at[p], sem_b[p]).wait()

            @pl.when(j >= 2)
            def _():
                pltpu.make_async_copy(
                    msg.at[p], acc.at[idx.at[q, 1]], sem_s[p]).wait()

            @pl.when(j + 2 < nchunk)
            def _():
                pltpu.async_copy(sdx_hbm.at[wid, j + 2], idx.at[q2], sem_i[p])

            def edge(e, _c):
                h0 = buf_a[p, e, pl.ds(0, 16)] + buf_b[p, e, pl.ds(0, 16)]
                h1 = buf_a[p, e, pl.ds(16, 16)] + buf_b[p, e, pl.ds(16, 16)]
                a0 = jnp.maximum(h0, h0 * 0.01)
                a1 = jnp.maximum(h1, h1 * 0.01)
                m = b2_reg
                for k in range(16):
                    m = m + a0[k] * w2_rows[k]
                for k in range(16):
                    m = m + a1[k] * w2_rows[16 + k]
                e2 = jnp.exp(-2.0 * jnp.abs(m))
                t = (1.0 - e2) / (1.0 + e2)
                msg[p, e, :] = jnp.where(m < 0.0, -t, t)
                return 0

            lax.fori_loop(0, C, edge, 0)
            pltpu.async_copy(
                msg.at[p], acc.at[idx.at[q, 1]], sem_s[p], add=True)

            @pl.when(j + 2 < nchunk)
            def _():
                pltpu.make_async_copy(
                    sdx_hbm.at[wid, j + 2], idx.at[q2], sem_i[p]).wait()
                pltpu.async_copy(
                    a_hbm.at[idx.at[q2, 0]], buf_a.at[p], sem_a[p])
                pltpu.async_copy(
                    b_hbm.at[idx.at[q2, 1]], buf_b.at[p], sem_b[p])
        return 0

    lax.fori_loop(0, nchunk // 4, chunk_quad, 0)
    for b in range(2):
        pltpu.make_async_copy(
            msg.at[b], acc.at[idx.at[b, 1]], sem_s[b]).wait()
    plsc.subcore_barrier()

    rows8 = (n // NS) // 8 * 8
    base = s * rows8
    pltpu.sync_copy(
        acc.at[pl.ds(base, rows8)], out_hbm.at[c, pl.ds(base, rows8)])
    rem = n - NS * rows8

    @pl.when(s == NS - 1)
    def _():
        pltpu.sync_copy(acc.at[pl.ds(NS * rows8, rem)],
                        out_hbm.at[c, pl.ds(NS * rows8, rem)])


def _stage2(a, bd, sdx, w2, b2, n, nchunk, acc_rows):
    mesh = plsc.VectorSubcoreMesh(core_axis_name="c", subcore_axis_name="s")
    body = functools.partial(_edge_body, nchunk, n, acc_rows)
    return pl.kernel(
        body,
        out_type=jax.ShapeDtypeStruct((NC, n, 16), jnp.float32),
        mesh=mesh,
        compiler_params=pltpu.CompilerParams(use_tc_tiling_on_sc=False),
        scratch_types=[
            pltpu.VMEM((4, 2, C), jnp.int32),
            pltpu.VMEM((2, C, 32), jnp.float32),
            pltpu.VMEM((2, C, 32), jnp.float32),
            pltpu.VMEM((2, C, 16), jnp.float32),
            pltpu.VMEM((32, 16), jnp.float32),
            pltpu.VMEM((16,), jnp.float32),
            pltpu.VMEM_SHARED((acc_rows, 16), jnp.float32),
            pltpu.SemaphoreType.DMA,
            pltpu.SemaphoreType.DMA,
            pltpu.SemaphoreType.DMA,
            pltpu.SemaphoreType.DMA,
            pltpu.SemaphoreType.DMA,
            pltpu.SemaphoreType.DMA,
            pltpu.SemaphoreType.DMA,
            pltpu.SemaphoreType.DMA,
        ],
    )(a, bd, sdx, w2, b2)


def _stage3(pos, h, u, partials, u1, ub1, u2, ub2):
    n = pos.shape[0]
    blk = 4000 if n % 4000 == 0 else n
    grid = n // blk

    def body(pos_ref, h_ref, u_ref, p_ref, u1_ref, ub1_ref, u2_ref, ub2_ref,
             o_ref):
        sum_h = p_ref[0] + p_ref[1]
        inp2 = jnp.concatenate(
            [pos_ref[...], h_ref[...], sum_h, u_ref[...]], axis=1)
        z = jnp.dot(inp2, u1_ref[...], preferred_element_type=jnp.float32)
        z = z + ub1_ref[...]
        z = jnp.maximum(z, 0.01 * z)
        o_ref[...] = jnp.tanh(
            jnp.dot(z, u2_ref[...], preferred_element_type=jnp.float32)
            + ub2_ref[...])

    return pl.pallas_call(
        body,
        grid=(grid,),
        in_specs=[
            pl.BlockSpec((blk, 2), lambda i: (i, 0)),
            pl.BlockSpec((blk, 16), lambda i: (i, 0)),
            pl.BlockSpec((blk, 8), lambda i: (i, 0)),
            pl.BlockSpec((NC, blk, 16), lambda i: (0, i, 0)),
            pl.BlockSpec(u1.shape, lambda i: (0, 0)),
            pl.BlockSpec((1, 32), lambda i: (0, 0)),
            pl.BlockSpec(u2.shape, lambda i: (0, 0)),
            pl.BlockSpec((1, 16), lambda i: (0, 0)),
        ],
        out_specs=pl.BlockSpec((blk, 16), lambda i: (i, 0)),
        out_shape=jax.ShapeDtypeStruct((n, 16), jnp.float32),
    )(pos, h, u, partials, u1, ub1, u2, ub2)


def kernel(pos, h, u, edge_index, W1, b1, W2, b2, U1, ub1, U2, ub2):
    n = pos.shape[0]
    e = edge_index.shape[1]

    x1 = jnp.concatenate([pos, u, h], axis=1)
    w1s = jnp.concatenate([W1[0:2], W1[4:28]], axis=0)
    w1d = W1[2:4]
    a, bd = _stage1(x1, pos, w1s, w1d, b1.reshape(1, 32))

    nchunk = 4 * -(-e // (NW * C * 4))
    e_pad = NW * nchunk * C
    src = edge_index[0].astype(jnp.int32)
    dst = edge_index[1].astype(jnp.int32)
    src_p = jnp.concatenate(
        [src, jnp.zeros((e_pad - e,), jnp.int32)]).reshape(NW, nchunk, 1, C)
    dst_p = jnp.concatenate(
        [dst, jnp.full((e_pad - e,), n, jnp.int32)]).reshape(NW, nchunk, 1, C)
    sdx = jnp.concatenate([src_p, dst_p], axis=2)

    acc_rows = NS * C * (-(-(n + 1) // (NS * C)))
    partials = _stage2(a, bd, sdx, W2, b2, n, nchunk, acc_rows)

    return _stage3(pos, h, u, partials, U1, ub1.reshape(1, 32), U2,
                   ub2.reshape(1, 16))
